# Initial kernel scaffold; baseline (speedup 1.0000x reference)
import jax
import jax.numpy as jnp
from jax.experimental import pallas as pl


def kernel(hidden_states, router_probs, top_k):
    raise NotImplementedError("write your pallas kernel here")


# speedup vs baseline: 2.0243x; 2.0243x over previous
import functools

import jax
import jax.numpy as jnp
from jax import lax
from jax.experimental import pallas as pl
from jax.experimental.pallas import tpu as pltpu
from jax.experimental.pallas import tpu_sc as plsc

_TB = 1024


def _top2(v):
    b, e = v.shape
    iota_e = lax.broadcasted_iota(jnp.int32, (b, e), 1)
    m1 = jnp.max(v, axis=1, keepdims=True)
    i1 = jnp.min(jnp.where(v == m1, iota_e, e), axis=1)
    v2 = jnp.where(iota_e == i1[:, None], -jnp.inf, v)
    m2 = jnp.max(v2, axis=1, keepdims=True)
    i2 = jnp.min(jnp.where(v2 == m2, iota_e, e), axis=1)
    return m1, i1, m2, i2, iota_e


def _hist_body(probs_ref, hist_ref, minmax_ref):
    v = probs_ref[...]
    b, e = v.shape
    _, i1, _, i2, iota_e = _top2(v)
    oh = (iota_e == i1[:, None]).astype(jnp.float32) + (
        iota_e == i2[:, None]
    ).astype(jnp.float32)
    hist_ref[...] = jnp.sum(oh.reshape(b // 64, 64, e), axis=1)
    minmax_ref[...] = jnp.concatenate(
        [jnp.min(v)[None, None], jnp.max(v)[None, None]], axis=1
    )[None]


def _dest_body(probs_ref, hist_ref, minmax_ref, ei_ref, rw_ref, de_ref, do_ref, start_s):
    b, e = probs_ref.shape
    g = b // 64
    gtot = hist_ref.shape[0]

    @pl.when(pl.program_id(0) == 0)
    def _scan():
        bh = hist_ref[...]
        x = bh
        k = 1
        while k < gtot:
            x = x + jnp.concatenate(
                [jnp.zeros((k, e), jnp.float32), x[:-k]], axis=0
            )
            k *= 2
        excl_grp = x - bh
        totals = x[gtot - 1 :, :]
        y = totals
        k = 1
        while k < e:
            y = y + jnp.concatenate(
                [jnp.zeros((1, k), jnp.float32), y[:, :-k]], axis=1
            )
            k *= 2
        start_s[...] = excl_grp + (y - totals)

    v = probs_ref[...]
    m1, i1, m2, i2, iota_e = _top2(v)

    mm = minmax_ref[...]
    needs_softmax = (jnp.min(mm[:, :, 0]) < 0.0) | (jnp.max(mm[:, :, 1]) > 1.0)
    v1s = m1[:, 0]
    v2s = m2[:, 0]
    e2v = jnp.exp(v2s - v1s)
    s = v1s + v2s
    w1 = jnp.where(needs_softmax, 1.0 / (1.0 + e2v), v1s / s)
    w2 = jnp.where(needs_softmax, e2v / (1.0 + e2v), v2s / s)
    ei_ref[...] = jnp.concatenate([i1[:, None], i2[:, None]], axis=1)
    rw_ref[...] = jnp.concatenate([w1[:, None], w2[:, None]], axis=1)

    oh1 = (iota_e == i1[:, None]).astype(jnp.float32)
    oh2 = (iota_e == i2[:, None]).astype(jnp.float32)
    ohs = (oh1 + oh2).reshape(g, 64, e)

    x = ohs
    k = 1
    while k < 64:
        x = x + jnp.concatenate(
            [jnp.zeros((g, k, e), jnp.float32), x[:, :-k, :]], axis=1
        )
        k *= 2
    excl_tok = x - ohs

    pid = pl.program_id(0)
    start = start_s[pl.ds(pid * g, g), :]
    slot = excl_tok + start[:, None, :]
    de = jnp.sum(slot * oh1.reshape(g, 64, e), axis=2)
    do = jnp.sum(slot * oh2.reshape(g, 64, e), axis=2) + (i1 == i2).astype(
        jnp.float32
    ).reshape(g, 64)
    de_ref[...] = de.astype(jnp.int32)
    do_ref[...] = do.astype(jnp.int32)


def _routing_tc(router_probs):
    b, e = router_probs.shape
    nblk = b // _TB
    gpb = _TB // 64
    gtot = b // 64
    hist, minmax = pl.pallas_call(
        _hist_body,
        grid=(nblk,),
        in_specs=[pl.BlockSpec((_TB, e), lambda i: (i, 0))],
        out_specs=[
            pl.BlockSpec((gpb, e), lambda i: (i, 0)),
            pl.BlockSpec((1, 1, 2), lambda i: (i, 0, 0)),
        ],
        out_shape=[
            jax.ShapeDtypeStruct((gtot, e), jnp.float32),
            jax.ShapeDtypeStruct((nblk, 1, 2), jnp.float32),
        ],
    )(router_probs)
    return pl.pallas_call(
        _dest_body,
        grid=(nblk,),
        in_specs=[
            pl.BlockSpec((_TB, e), lambda i: (i, 0)),
            pl.BlockSpec((gtot, e), lambda i: (0, 0)),
            pl.BlockSpec((nblk, 1, 2), lambda i: (0, 0, 0)),
        ],
        out_specs=[
            pl.BlockSpec((_TB, 2), lambda i: (i, 0)),
            pl.BlockSpec((_TB, 2), lambda i: (i, 0)),
            pl.BlockSpec((gpb, 64), lambda i: (i, 0)),
            pl.BlockSpec((gpb, 64), lambda i: (i, 0)),
        ],
        out_shape=[
            jax.ShapeDtypeStruct((b, 2), jnp.int32),
            jax.ShapeDtypeStruct((b, 2), jnp.float32),
            jax.ShapeDtypeStruct((gtot, 64), jnp.int32),
            jax.ShapeDtypeStruct((gtot, 64), jnp.int32),
        ],
        scratch_shapes=[pltpu.VMEM((gtot, e), jnp.float32)],
    )(router_probs, hist, minmax)


def _make_dispatch(b, d):
    nw = 32
    ch_t = b // nw
    rb = 64
    nblk = ch_t // rb
    mesh = plsc.VectorSubcoreMesh(core_axis_name="c", subcore_axis_name="s")

    @functools.partial(
        pl.kernel,
        mesh=mesh,
        out_type=jax.ShapeDtypeStruct((2 * b, d), jnp.float32),
        scratch_types=[
            pltpu.VMEM((nblk, rb), jnp.int32),
            pltpu.VMEM((nblk, rb), jnp.int32),
            pltpu.VMEM((rb, d), jnp.float32),
            pltpu.SemaphoreType.DMA,
        ],
    )
    def dispatch(hid_hbm, de_hbm, do_hbm, out_hbm, de_v, do_v, rows_v, sem):
        wid = lax.axis_index("s") * 2 + lax.axis_index("c")
        g0 = wid * nblk
        pltpu.sync_copy(de_hbm.at[pl.ds(g0, nblk)], de_v)
        pltpu.sync_copy(do_hbm.at[pl.ds(g0, nblk)], do_v)
        tok0 = wid * ch_t

        def blk(kk, carry):
            pltpu.sync_copy(hid_hbm.at[pl.ds(tok0 + kk * rb, rb)], rows_v)
            cpe = pltpu.async_copy(rows_v, out_hbm.at[de_v.at[kk]], sem)
            cpo = pltpu.async_copy(rows_v, out_hbm.at[do_v.at[kk]], sem)
            cpe.wait()
            cpo.wait()
            return carry

        lax.fori_loop(0, nblk, blk, 0)

    return dispatch


def kernel(hidden_states, router_probs, top_k):
    b, d = hidden_states.shape
    expert_indices, routing_weights, de, do = _routing_tc(router_probs)
    dispatch = _make_dispatch(b, d)
    dispatched = dispatch(hidden_states, de, do)
    return dispatched, expert_indices, routing_weights
